# baseline (speedup 1.0000x reference)
import functools
import math

import jax
import jax.numpy as jnp
from jax import lax
from jax.experimental import pallas as pl
from jax.experimental.pallas import tpu as pltpu
from jax.experimental.pallas import tpu_sc as plsc

N = 10000
E = 320000
H = 128
HEADS = 4
B = 64

NCORE = 2
NSUB = 16
NT = NCORE * NSUB
CH = 128
NJUNK = 112
NACC = N + NJUNK
ZROWS = NACC // NSUB


def _pad_edges(idx, pad, junk_base):
    npad = pad - idx.shape[0]
    fill_mod = N if junk_base is None else NJUNK
    base = 0 if junk_base is None else junk_base
    fill = base + (jnp.arange(npad, dtype=jnp.int32) * 97) % fill_mod
    out = jnp.concatenate([idx, fill])
    return out.reshape(NT, -1, CH)


def _gin_sc_agg(h, idx_gin, zrows):
    nchunk = idx_gin.shape[1]
    mesh = plsc.VectorSubcoreMesh(core_axis_name="c", subcore_axis_name="s")

    @functools.partial(
        pl.kernel,
        out_type=jax.ShapeDtypeStruct((NCORE, N, H), jnp.float32),
        mesh=mesh,
        compiler_params=pltpu.CompilerParams(needs_layout_passes=False),
        scratch_types=[
            pltpu.VMEM_SHARED((NACC, H), jnp.float32),
            pltpu.VMEM((2, CH), jnp.int32),
            pltpu.VMEM((2, CH), jnp.int32),
            pltpu.VMEM((CH, H), jnp.float32),
            pltpu.VMEM((CH, H), jnp.float32),
            pltpu.SemaphoreType.DMA,
            pltpu.SemaphoreType.DMA,
            pltpu.SemaphoreType.DMA,
            pltpu.SemaphoreType.DMA,
        ],
    )
    def k(h_hbm, idx_hbm, z_hbm, out_hbm, acc, idx0, idx1, rows0, rows1,
          gsem0, gsem1, ssem0, ssem1):
        c = lax.axis_index("c")
        s = lax.axis_index("s")
        wid = c * NSUB + s

        pltpu.sync_copy(z_hbm, acc.at[pl.ds(s * ZROWS, ZROWS)])
        plsc.subcore_barrier()

        pltpu.sync_copy(idx_hbm.at[wid, 0], idx0)
        pltpu.async_copy(h_hbm.at[idx0.at[0]], rows0, gsem0)

        def half(j, idx_c, rows_c, gsem_c, ssem_c, idx_n, rows_n,
                 gsem_n, ssem_n):
            pltpu.make_async_copy(h_hbm.at[idx_c.at[0]], rows_c,
                                  gsem_c).wait()

            @pl.when(j < nchunk - 1)
            def _():
                @pl.when(j >= 1)
                def _():
                    pltpu.make_async_copy(rows_n, acc.at[idx_n.at[1]],
                                          ssem_n).wait()
                pltpu.sync_copy(idx_hbm.at[wid, j + 1], idx_n)
                pltpu.async_copy(h_hbm.at[idx_n.at[0]], rows_n, gsem_n)

            pltpu.async_copy(rows_c, acc.at[idx_c.at[1]], add=True,
                             sem=ssem_c)

        def body(p, _):
            j = p * 2
            half(j, idx0, rows0, gsem0, ssem0, idx1, rows1, gsem1, ssem1)
            half(j + 1, idx1, rows1, gsem1, ssem1, idx0, rows0, gsem0, ssem0)
            return 0

        lax.fori_loop(0, nchunk // 2, body, 0)
        if nchunk % 2 == 1:
            half(jnp.int32(nchunk - 1), idx0, rows0, gsem0, ssem0,
                 idx1, rows1, gsem1, ssem1)
            pltpu.make_async_copy(rows0, acc.at[idx0.at[1]], ssem0).wait()
            pltpu.make_async_copy(rows1, acc.at[idx1.at[1]], ssem1).wait()
        else:
            pltpu.make_async_copy(rows0, acc.at[idx0.at[1]], ssem0).wait()
            pltpu.make_async_copy(rows1, acc.at[idx1.at[1]], ssem1).wait()
        plsc.subcore_barrier()

        @pl.when(s < NSUB - 1)
        def _():
            pltpu.sync_copy(acc.at[pl.ds(s * ZROWS, ZROWS)],
                            out_hbm.at[c, pl.ds(s * ZROWS, ZROWS)])

        @pl.when(s == NSUB - 1)
        def _():
            last = N - (NSUB - 1) * ZROWS
            pltpu.sync_copy(acc.at[pl.ds((NSUB - 1) * ZROWS, last)],
                            out_hbm.at[c, pl.ds((NSUB - 1) * ZROWS, last)])

    return k(h, idx_gin, zrows)


HW = 144


def _gat_sc(xp_flat, ad8, idx_gat, zrows_hw):
    nc = idx_gat.shape[2]
    mesh = plsc.VectorSubcoreMesh(core_axis_name="c", subcore_axis_name="s")

    @functools.partial(
        pl.kernel,
        out_type=jax.ShapeDtypeStruct((HEADS, N, HW), jnp.float32),
        mesh=mesh,
        compiler_params=pltpu.CompilerParams(needs_layout_passes=False,
                                             use_tc_tiling_on_sc=False),
        scratch_types=[
            pltpu.VMEM_SHARED((NACC, HW), jnp.float32),
            pltpu.VMEM((3, CH), jnp.int32),
            pltpu.VMEM((3, CH), jnp.int32),
            pltpu.VMEM((CH, 8), jnp.float32),
            pltpu.VMEM((CH + 16,), jnp.float32),
            pltpu.VMEM((CH, HW), jnp.float32),
            pltpu.VMEM((CH, HW), jnp.float32),
            pltpu.SemaphoreType.DMA,
            pltpu.SemaphoreType.DMA,
            pltpu.SemaphoreType.DMA,
            pltpu.SemaphoreType.DMA,
        ],
    )
    def k(xp_hbm, ad_hbm, idx_hbm, z_hbm, out_hbm,
          acc, idx0, idx1, adb, wbuf, rows0, rows1, gsem0, gsem1,
          ssem0, ssem1):
        c = lax.axis_index("c")
        s = lax.axis_index("s")
        col_as = jnp.full((16,), H + 1, jnp.int32)
        col0 = jnp.zeros((16,), jnp.int32)

        def head_pass(hp, _):
            head = c * (HEADS // NCORE) + hp
            pltpu.sync_copy(z_hbm, acc.at[pl.ds(s * ZROWS, ZROWS)])
            plsc.subcore_barrier()

            pltpu.sync_copy(idx_hbm.at[head, s, 0], idx0)
            pltpu.async_copy(xp_hbm.at[idx0.at[0]], rows0, gsem0)
            pltpu.async_copy(ad_hbm.at[idx0.at[1]], adb, gsem0)

            def half(j, idx_c, rows_c, gsem_c, ssem_c, idx_n, rows_n,
                     gsem_n, ssem_n):
                pltpu.make_async_copy(ad_hbm.at[idx_c.at[1]], adb,
                                      gsem_c).wait()
                pltpu.make_async_copy(xp_hbm.at[idx_c.at[0]], rows_c,
                                      gsem_c).wait()
                for i in range(CH // 16):
                    e16 = lax.iota(jnp.int32, 16) + (i * 16)
                    logit = (plsc.load_gather(rows_c, [e16, col_as])
                             + plsc.load_gather(adb, [e16, col0]))
                    logit = jnp.maximum(logit, 0.2 * logit)
                    wbuf[pl.ds(i * 16, 16)] = jnp.exp(logit)

                @pl.when(j < nc - 1)
                def _():
                    @pl.when(j >= 1)
                    def _():
                        pltpu.make_async_copy(rows_n, acc.at[idx_n.at[2]],
                                              ssem_n).wait()
                    pltpu.sync_copy(idx_hbm.at[head, s, j + 1], idx_n)
                    pltpu.async_copy(xp_hbm.at[idx_n.at[0]], rows_n, gsem_n)
                    pltpu.async_copy(ad_hbm.at[idx_n.at[1]], adb, gsem_n)

                def scale(p2, _):
                    e = p2 * 2
                    w0 = wbuf[pl.ds(e, 16)][0]
                    w1 = wbuf[pl.ds(e + 1, 16)][0]
                    for q in range(HW // 16):
                        rows_c[e, pl.ds(q * 16, 16)] = (
                            rows_c[e, pl.ds(q * 16, 16)] * w0)
                    for q in range(HW // 16):
                        rows_c[e + 1, pl.ds(q * 16, 16)] = (
                            rows_c[e + 1, pl.ds(q * 16, 16)] * w1)
                    return 0

                lax.fori_loop(0, CH // 2, scale, 0)
                pltpu.async_copy(rows_c, acc.at[idx_c.at[2]], add=True,
                                 sem=ssem_c)

            def chunk_pair(p, _):
                j = p * 2
                half(j, idx0, rows0, gsem0, ssem0, idx1, rows1, gsem1, ssem1)
                half(j + 1, idx1, rows1, gsem1, ssem1, idx0, rows0, gsem0,
                     ssem0)
                return 0

            lax.fori_loop(0, nc // 2, chunk_pair, 0)
            pltpu.make_async_copy(rows0, acc.at[idx0.at[2]], ssem0).wait()
            pltpu.make_async_copy(rows1, acc.at[idx1.at[2]], ssem1).wait()
            plsc.subcore_barrier()

            @pl.when(s < NSUB - 1)
            def _():
                pltpu.sync_copy(acc.at[pl.ds(s * ZROWS, ZROWS)],
                                out_hbm.at[head, pl.ds(s * ZROWS, ZROWS)])

            @pl.when(s == NSUB - 1)
            def _():
                last = N - (NSUB - 1) * ZROWS
                pltpu.sync_copy(acc.at[pl.ds((NSUB - 1) * ZROWS, last)],
                                out_hbm.at[head, pl.ds((NSUB - 1) * ZROWS,
                                                       last)])
            plsc.subcore_barrier()
            return 0

        lax.fori_loop(0, HEADS // NCORE, head_pass, 0)

    return k(xp_flat, ad8, idx_gat, zrows_hw)


_BBN = 1000


def _gat_build_kernel(x_ref, w_ref, as_ref, ad_ref, aug_ref, ad_o):
    xp = x_ref[...] @ w_ref[...]
    a_s = jnp.sum(xp * as_ref[0], axis=-1)
    aug_ref[0] = jnp.concatenate(
        [xp, jnp.ones((N, 1), jnp.float32), a_s[:, None],
         jnp.zeros((N, HW - H - 2), jnp.float32)], axis=1)
    ad_o[0, 0] = jnp.sum(xp * ad_ref[0], axis=-1)


def _gat_build_tc(x, W, a_src, a_dst):
    f = x.shape[1]
    aug, a_d3 = pl.pallas_call(
        _gat_build_kernel,
        grid=(HEADS,),
        in_specs=[
            pl.BlockSpec((N, f), lambda h: (0, 0)),
            pl.BlockSpec((f, H), lambda h: (0, h)),
            pl.BlockSpec((1, 1, H), lambda h: (h, 0, 0)),
            pl.BlockSpec((1, 1, H), lambda h: (h, 0, 0)),
        ],
        out_specs=[
            pl.BlockSpec((1, N, HW), lambda h: (h, 0, 0)),
            pl.BlockSpec((1, 1, N), lambda h: (h, 0, 0)),
        ],
        out_shape=[
            jax.ShapeDtypeStruct((HEADS, N, HW), jnp.float32),
            jax.ShapeDtypeStruct((HEADS, 1, N), jnp.float32),
        ],
    )(x, W, a_src.reshape(HEADS, 1, H), a_dst.reshape(HEADS, 1, H))
    return aug, a_d3


def _gat_finish_kernel(g_ref, bg_ref, out_ref):
    acc4 = g_ref[...]
    num = acc4[:, :, :H]
    den = acc4[:, :, H:H + 1]
    out_ref[...] = jnp.maximum((num / den).mean(0) + bg_ref[...], 0.0)


def _gat_finish_tc(gacc, bg):
    return pl.pallas_call(
        _gat_finish_kernel,
        grid=(N // _BBN,),
        in_specs=[
            pl.BlockSpec((HEADS, _BBN, HW), lambda i: (0, i, 0)),
            pl.BlockSpec((H,), lambda i: (0,)),
        ],
        out_specs=pl.BlockSpec((_BBN, H), lambda i: (i, 0)),
        out_shape=jax.ShapeDtypeStruct((N, H), jnp.float32),
    )(gacc, bg)


def _gat_sc_full(x, W, a_src, a_dst, bg, idx_gat, zrows_hw):
    aug, a_d3 = _gat_build_tc(x, W, a_src, a_dst)
    a_dp = jnp.pad(a_d3.reshape(HEADS, N), ((0, 0), (0, NJUNK)))
    ad8 = jnp.pad(a_dp.reshape(HEADS * NACC, 1), ((0, 0), (0, 7)))
    gacc = _gat_sc(aug.reshape(HEADS * N, HW), ad8, idx_gat, zrows_hw)
    return _gat_finish_tc(gacc, bg)


_GBN = 1000


def _gin_block_kernel(h_ref, p_ref, w1_ref, b1_ref, w2_ref, b2_ref,
                      g_ref, bb_ref, out_ref):
    h = h_ref[...]
    agg = p_ref[0] + p_ref[1]
    t = h + agg
    t = jnp.maximum(t @ w1_ref[...] + b1_ref[...], 0.0)
    t = h + (t @ w2_ref[...] + b2_ref[...])
    m = t.mean(-1, keepdims=True)
    v = ((t - m) ** 2).mean(-1, keepdims=True)
    out_ref[...] = (t - m) / jnp.sqrt(v + 1e-5) * g_ref[...] + bb_ref[...]


def _gin_block_tc(h, partials, w1, b1, w2, b2, g, b):
    return pl.pallas_call(
        _gin_block_kernel,
        grid=(N // _GBN,),
        in_specs=[
            pl.BlockSpec((_GBN, H), lambda i: (i, 0)),
            pl.BlockSpec((NCORE, _GBN, H), lambda i: (0, i, 0)),
            pl.BlockSpec((H, H), lambda i: (0, 0)),
            pl.BlockSpec((H,), lambda i: (0,)),
            pl.BlockSpec((H, H), lambda i: (0, 0)),
            pl.BlockSpec((H,), lambda i: (0,)),
            pl.BlockSpec((H,), lambda i: (0,)),
            pl.BlockSpec((H,), lambda i: (0,)),
        ],
        out_specs=pl.BlockSpec((_GBN, H), lambda i: (i, 0)),
        out_shape=jax.ShapeDtypeStruct((N, H), jnp.float32),
    )(h, partials, w1, b1, w2, b2, g, b)


def _seg_softmax(logits, segs, num):
    mx = jax.ops.segment_max(logits, segs, num_segments=num)
    mx = jnp.where(jnp.isfinite(mx), mx, 0.0)
    e = jnp.exp(logits - mx[segs])
    s = jax.ops.segment_sum(e, segs, num_segments=num)
    return e / (s[segs] + 1e-16)


def _gat(x, src, dst, W, a_src, a_dst, bias):
    xp = (x @ W).reshape(N, HEADS, H)
    a_s = jnp.sum(xp * a_src[None], -1)
    a_d = jnp.sum(xp * a_dst[None], -1)
    alpha = jax.nn.leaky_relu(a_s[src] + a_d[dst], 0.2)
    alpha = _seg_softmax(alpha, dst, N)
    out = jax.ops.segment_sum(xp[src] * alpha[..., None], dst, num_segments=N)
    return out.mean(1) + bias


def _gin(x, src, dst, w1, b1, w2, b2):
    agg = jax.ops.segment_sum(x[src], dst, num_segments=N)
    h = x + agg
    h = jax.nn.relu(h @ w1 + b1)
    return h @ w2 + b2


def _layer_norm(x, g, b):
    m = x.mean(-1, keepdims=True)
    v = ((x - m) ** 2).mean(-1, keepdims=True)
    return (x - m) / jnp.sqrt(v + 1e-5) * g + b


_BN = 1000


def _pool_kernel(h2_ref, bat_ref, gw1_ref, gb1_ref, gw2_ref, gb2_ref,
                 pooled_ref, s_ref):
    i = pl.program_id(0)

    h2 = h2_ref[...]
    g = jnp.maximum(h2 @ gw1_ref[...] + gb1_ref[...], 0.0)
    logit = g @ gw2_ref[...] + gb2_ref[...]
    e = jnp.exp(logit[:, 0])

    bat = bat_ref[...]
    onehot = (bat == lax.broadcasted_iota(jnp.int32, (1, B), 1)).astype(
        jnp.float32)

    pooled_blk = onehot.T @ (e[:, None] * h2)
    s_blk = onehot.T @ e[:, None]

    @pl.when(i == 0)
    def _():
        pooled_ref[...] = jnp.zeros_like(pooled_ref)
        s_ref[...] = jnp.zeros_like(s_ref)

    pooled_ref[...] += pooled_blk
    s_ref[...] += s_blk


def _final_kernel(pooled_ref, s_ref, l1w_ref, l1b_ref, lnfg_ref, lnfb_ref,
                  l2w_ref, l2b_ref, out_ref):
    pooled = pooled_ref[...] / (s_ref[...] + 1e-16)
    z = pooled @ l1w_ref[...] + l1b_ref[...]
    m = z.mean(-1, keepdims=True)
    v = ((z - m) ** 2).mean(-1, keepdims=True)
    z = (z - m) / jnp.sqrt(v + 1e-5) * lnfg_ref[...] + lnfb_ref[...]
    z = jnp.maximum(z, 0.0)
    out_ref[...] = z @ l2w_ref[...] + l2b_ref[...]


def _pool_and_head(h2, batch, gw1, gb1, gw2, gb2, l1w, l1b, lnfg, lnfb,
                   l2w, l2b):
    bat2 = batch[:, None]
    pooled, s = pl.pallas_call(
        _pool_kernel,
        grid=(N // _BN,),
        in_specs=[
            pl.BlockSpec((_BN, H), lambda i: (i, 0)),
            pl.BlockSpec((_BN, 1), lambda i: (i, 0)),
            pl.BlockSpec((H, H), lambda i: (0, 0)),
            pl.BlockSpec((H,), lambda i: (0,)),
            pl.BlockSpec((H, 1), lambda i: (0, 0)),
            pl.BlockSpec((1,), lambda i: (0,)),
        ],
        out_specs=[
            pl.BlockSpec((B, H), lambda i: (0, 0)),
            pl.BlockSpec((B, 1), lambda i: (0, 0)),
        ],
        out_shape=[
            jax.ShapeDtypeStruct((B, H), jnp.float32),
            jax.ShapeDtypeStruct((B, 1), jnp.float32),
        ],
    )(h2, bat2, gw1, gb1, gw2, gb2)

    out = pl.pallas_call(
        _final_kernel,
        out_shape=jax.ShapeDtypeStruct((B, 6), jnp.float32),
    )(pooled, s, l1w, l1b, lnfg, lnfb, l2w, l2b)
    return out


def kernel(x, edge_index, batch, W1, as1, ad1, bg1, g1w1, g1b1, g1w2, g1b2,
           ln1g, ln1b, W2, as2, ad2, bg2, g2w1, g2b1, g2w2, g2b2, ln2g, ln2b,
           gw1, gb1, gw2, gb2, l1w, l1b, lnfg, lnfb, l2w, l2b):
    src0, dst0 = edge_index[0], edge_index[1]
    loop = jnp.arange(N, dtype=edge_index.dtype)
    src = jnp.concatenate([src0, loop])
    dst = jnp.concatenate([dst0, loop])

    e_pad = NT * CH * math.ceil(E / (NT * CH))
    srcp = _pad_edges(src0, e_pad, None)
    dstp = _pad_edges(dst0, e_pad, N)
    idx_gin = jnp.stack([srcp, dstp], axis=2)
    zrows = jnp.zeros((ZROWS, H), jnp.float32)
    zrows_hw = jnp.zeros((ZROWS, HW), jnp.float32)

    eg_pad = NSUB * CH * math.ceil((E + N) / (NSUB * CH))
    npad_g = eg_pad - (E + N)
    src_g = jnp.concatenate(
        [src, (jnp.arange(npad_g, dtype=jnp.int32) * 97) % N])
    dst_g = jnp.concatenate(
        [dst, N + (jnp.arange(npad_g, dtype=jnp.int32) * 97) % NJUNK])
    src_t = src_g.reshape(NSUB, -1, CH)
    dst_t = dst_g.reshape(NSUB, -1, CH)
    idx_gat = jnp.stack(
        [jnp.stack([src_t + hd * N, dst_t + hd * NACC, dst_t], axis=2)
         for hd in range(HEADS)], axis=0)

    h = _gat_sc_full(x, W1, as1, ad1, bg1, idx_gat, zrows_hw)
    h = _gin_block_tc(h, _gin_sc_agg(h, idx_gin, zrows),
                      g1w1, g1b1, g1w2, g1b2, ln1g, ln1b)
    h2 = _gat_sc_full(h, W2, as2, ad2, bg2, idx_gat, zrows_hw)
    h2 = _gin_block_tc(h2, _gin_sc_agg(h2, idx_gin, zrows),
                       g2w1, g2b1, g2w2, g2b2, ln2g, ln2b)

    return _pool_and_head(h2, batch, gw1, gb1, gw2, gb2, l1w, l1b,
                          lnfg, lnfb, l2w, l2b)


# speedup vs baseline: 1.0006x; 1.0006x over previous
import functools
import math

import jax
import jax.numpy as jnp
from jax import lax
from jax.experimental import pallas as pl
from jax.experimental.pallas import tpu as pltpu
from jax.experimental.pallas import tpu_sc as plsc

N = 10000
E = 320000
H = 128
HEADS = 4
B = 64

NCORE = 2
NSUB = 16
NT = NCORE * NSUB
CH = 128
NJUNK = 112
NACC = N + NJUNK
ZROWS = NACC // NSUB


def _pad_edges(idx, pad, junk_base):
    npad = pad - idx.shape[0]
    fill_mod = N if junk_base is None else NJUNK
    base = 0 if junk_base is None else junk_base
    fill = base + (jnp.arange(npad, dtype=jnp.int32) * 97) % fill_mod
    out = jnp.concatenate([idx, fill])
    return out.reshape(NT, -1, CH)


def _gin_sc_agg(h, idx_gin, zrows):
    nchunk = idx_gin.shape[1]
    mesh = plsc.VectorSubcoreMesh(core_axis_name="c", subcore_axis_name="s")

    @functools.partial(
        pl.kernel,
        out_type=jax.ShapeDtypeStruct((NCORE, N, H), jnp.float32),
        mesh=mesh,
        compiler_params=pltpu.CompilerParams(needs_layout_passes=False),
        scratch_types=[
            pltpu.VMEM_SHARED((NACC, H), jnp.float32),
            pltpu.VMEM((2, CH), jnp.int32),
            pltpu.VMEM((2, CH), jnp.int32),
            pltpu.VMEM((CH, H), jnp.float32),
            pltpu.VMEM((CH, H), jnp.float32),
            pltpu.SemaphoreType.DMA,
            pltpu.SemaphoreType.DMA,
            pltpu.SemaphoreType.DMA,
            pltpu.SemaphoreType.DMA,
        ],
    )
    def k(h_hbm, idx_hbm, z_hbm, out_hbm, acc, idx0, idx1, rows0, rows1,
          gsem0, gsem1, ssem0, ssem1):
        c = lax.axis_index("c")
        s = lax.axis_index("s")
        wid = c * NSUB + s

        pltpu.sync_copy(z_hbm, acc.at[pl.ds(s * ZROWS, ZROWS)])
        plsc.subcore_barrier()

        pltpu.sync_copy(idx_hbm.at[wid, 0], idx0)
        pltpu.async_copy(h_hbm.at[idx0.at[0]], rows0, gsem0)

        def half(j, idx_c, rows_c, gsem_c, ssem_c, idx_n, rows_n,
                 gsem_n, ssem_n):
            pltpu.make_async_copy(h_hbm.at[idx_c.at[0]], rows_c,
                                  gsem_c).wait()

            @pl.when(j < nchunk - 1)
            def _():
                @pl.when(j >= 1)
                def _():
                    pltpu.make_async_copy(rows_n, acc.at[idx_n.at[1]],
                                          ssem_n).wait()
                pltpu.sync_copy(idx_hbm.at[wid, j + 1], idx_n)
                pltpu.async_copy(h_hbm.at[idx_n.at[0]], rows_n, gsem_n)

            pltpu.async_copy(rows_c, acc.at[idx_c.at[1]], add=True,
                             sem=ssem_c)

        def body(p, _):
            j = p * 2
            half(j, idx0, rows0, gsem0, ssem0, idx1, rows1, gsem1, ssem1)
            half(j + 1, idx1, rows1, gsem1, ssem1, idx0, rows0, gsem0, ssem0)
            return 0

        lax.fori_loop(0, nchunk // 2, body, 0)
        if nchunk % 2 == 1:
            half(jnp.int32(nchunk - 1), idx0, rows0, gsem0, ssem0,
                 idx1, rows1, gsem1, ssem1)
            pltpu.make_async_copy(rows0, acc.at[idx0.at[1]], ssem0).wait()
            pltpu.make_async_copy(rows1, acc.at[idx1.at[1]], ssem1).wait()
        else:
            pltpu.make_async_copy(rows0, acc.at[idx0.at[1]], ssem0).wait()
            pltpu.make_async_copy(rows1, acc.at[idx1.at[1]], ssem1).wait()
        plsc.subcore_barrier()

        @pl.when(s < NSUB - 1)
        def _():
            pltpu.sync_copy(acc.at[pl.ds(s * ZROWS, ZROWS)],
                            out_hbm.at[c, pl.ds(s * ZROWS, ZROWS)])

        @pl.when(s == NSUB - 1)
        def _():
            last = N - (NSUB - 1) * ZROWS
            pltpu.sync_copy(acc.at[pl.ds((NSUB - 1) * ZROWS, last)],
                            out_hbm.at[c, pl.ds((NSUB - 1) * ZROWS, last)])

    return k(h, idx_gin, zrows)


HW = 144


def _gat_sc(xp_flat, ad8, idx_gat, zrows_hw):
    nc = idx_gat.shape[2]
    mesh = plsc.VectorSubcoreMesh(core_axis_name="c", subcore_axis_name="s")

    @functools.partial(
        pl.kernel,
        out_type=jax.ShapeDtypeStruct((HEADS, N, HW), jnp.float32),
        mesh=mesh,
        compiler_params=pltpu.CompilerParams(needs_layout_passes=False,
                                             use_tc_tiling_on_sc=False),
        scratch_types=[
            pltpu.VMEM_SHARED((NACC, HW), jnp.float32),
            pltpu.VMEM((3, CH), jnp.int32),
            pltpu.VMEM((3, CH), jnp.int32),
            pltpu.VMEM((CH, 8), jnp.float32),
            pltpu.VMEM((CH + 16,), jnp.float32),
            pltpu.VMEM((CH, HW), jnp.float32),
            pltpu.VMEM((CH, HW), jnp.float32),
            pltpu.SemaphoreType.DMA,
            pltpu.SemaphoreType.DMA,
            pltpu.SemaphoreType.DMA,
            pltpu.SemaphoreType.DMA,
        ],
    )
    def k(xp_hbm, ad_hbm, idx_hbm, z_hbm, out_hbm,
          acc, idx0, idx1, adb, wbuf, rows0, rows1, gsem0, gsem1,
          ssem0, ssem1):
        c = lax.axis_index("c")
        s = lax.axis_index("s")
        col_as = jnp.full((16,), H + 1, jnp.int32)
        col0 = jnp.zeros((16,), jnp.int32)

        def head_pass(hp, _):
            head = c * (HEADS // NCORE) + hp
            pltpu.sync_copy(z_hbm, acc.at[pl.ds(s * ZROWS, ZROWS)])
            plsc.subcore_barrier()

            pltpu.sync_copy(idx_hbm.at[head, s, 0], idx0)
            pltpu.async_copy(xp_hbm.at[idx0.at[0]], rows0, gsem0)
            pltpu.async_copy(ad_hbm.at[idx0.at[1]], adb, gsem0)

            def half(j, idx_c, rows_c, gsem_c, ssem_c, idx_n, rows_n,
                     gsem_n, ssem_n):
                pltpu.make_async_copy(ad_hbm.at[idx_c.at[1]], adb,
                                      gsem_c).wait()
                pltpu.make_async_copy(xp_hbm.at[idx_c.at[0]], rows_c,
                                      gsem_c).wait()
                for i in range(CH // 16):
                    e16 = lax.iota(jnp.int32, 16) + (i * 16)
                    logit = (plsc.load_gather(rows_c, [e16, col_as])
                             + plsc.load_gather(adb, [e16, col0]))
                    logit = jnp.maximum(logit, 0.2 * logit)
                    wbuf[pl.ds(i * 16, 16)] = jnp.exp(logit)

                @pl.when(j < nc - 1)
                def _():
                    @pl.when(j >= 1)
                    def _():
                        pltpu.make_async_copy(rows_n, acc.at[idx_n.at[2]],
                                              ssem_n).wait()
                    pltpu.sync_copy(idx_hbm.at[head, s, j + 1], idx_n)
                    pltpu.async_copy(xp_hbm.at[idx_n.at[0]], rows_n, gsem_n)
                    pltpu.async_copy(ad_hbm.at[idx_n.at[1]], adb, gsem_n)

                def scale(p2, _):
                    e = p2 * 2
                    w0 = wbuf[pl.ds(e, 16)][0]
                    w1 = wbuf[pl.ds(e + 1, 16)][0]
                    for q in range(HW // 16):
                        rows_c[e, pl.ds(q * 16, 16)] = (
                            rows_c[e, pl.ds(q * 16, 16)] * w0)
                    for q in range(HW // 16):
                        rows_c[e + 1, pl.ds(q * 16, 16)] = (
                            rows_c[e + 1, pl.ds(q * 16, 16)] * w1)
                    return 0

                lax.fori_loop(0, CH // 2, scale, 0)
                pltpu.async_copy(rows_c, acc.at[idx_c.at[2]], add=True,
                                 sem=ssem_c)

            def chunk_pair(p, _):
                j = p * 2
                half(j, idx0, rows0, gsem0, ssem0, idx1, rows1, gsem1, ssem1)
                half(j + 1, idx1, rows1, gsem1, ssem1, idx0, rows0, gsem0,
                     ssem0)
                return 0

            lax.fori_loop(0, nc // 2, chunk_pair, 0)
            pltpu.make_async_copy(rows0, acc.at[idx0.at[2]], ssem0).wait()
            pltpu.make_async_copy(rows1, acc.at[idx1.at[2]], ssem1).wait()
            plsc.subcore_barrier()

            @pl.when(s < NSUB - 1)
            def _():
                pltpu.sync_copy(acc.at[pl.ds(s * ZROWS, ZROWS)],
                                out_hbm.at[head, pl.ds(s * ZROWS, ZROWS)])

            @pl.when(s == NSUB - 1)
            def _():
                last = N - (NSUB - 1) * ZROWS
                pltpu.sync_copy(acc.at[pl.ds((NSUB - 1) * ZROWS, last)],
                                out_hbm.at[head, pl.ds((NSUB - 1) * ZROWS,
                                                       last)])
            plsc.subcore_barrier()
            return 0

        lax.fori_loop(0, HEADS // NCORE, head_pass, 0)

    return k(xp_flat, ad8, idx_gat, zrows_hw)


_BBN = 1000


def _gat_build_kernel(x_ref, w_ref, as_ref, ad_ref, aug_ref, ad_o):
    xp = x_ref[...] @ w_ref[...]
    a_s = jnp.sum(xp * as_ref[0], axis=-1)
    aug_ref[0] = jnp.concatenate(
        [xp, jnp.ones((N, 1), jnp.float32), a_s[:, None],
         jnp.zeros((N, HW - H - 2), jnp.float32)], axis=1)
    ad_o[0, 0] = jnp.sum(xp * ad_ref[0], axis=-1)


def _gat_build_tc(x, W, a_src, a_dst):
    f = x.shape[1]
    aug, a_d3 = pl.pallas_call(
        _gat_build_kernel,
        grid=(HEADS,),
        in_specs=[
            pl.BlockSpec((N, f), lambda h: (0, 0)),
            pl.BlockSpec((f, H), lambda h: (0, h)),
            pl.BlockSpec((1, 1, H), lambda h: (h, 0, 0)),
            pl.BlockSpec((1, 1, H), lambda h: (h, 0, 0)),
        ],
        out_specs=[
            pl.BlockSpec((1, N, HW), lambda h: (h, 0, 0)),
            pl.BlockSpec((1, 1, N), lambda h: (h, 0, 0)),
        ],
        out_shape=[
            jax.ShapeDtypeStruct((HEADS, N, HW), jnp.float32),
            jax.ShapeDtypeStruct((HEADS, 1, N), jnp.float32),
        ],
    )(x, W, a_src.reshape(HEADS, 1, H), a_dst.reshape(HEADS, 1, H))
    return aug, a_d3


def _gat_finish_kernel(g_ref, bg_ref, out_ref):
    acc4 = g_ref[...]
    num = acc4[:, :, :H]
    den = acc4[:, :, H:H + 1]
    out_ref[...] = jnp.maximum((num / den).mean(0) + bg_ref[...], 0.0)


def _gat_finish_tc(gacc, bg):
    return pl.pallas_call(
        _gat_finish_kernel,
        grid=(N // _BBN,),
        in_specs=[
            pl.BlockSpec((HEADS, _BBN, HW), lambda i: (0, i, 0)),
            pl.BlockSpec((H,), lambda i: (0,)),
        ],
        out_specs=pl.BlockSpec((_BBN, H), lambda i: (i, 0)),
        out_shape=jax.ShapeDtypeStruct((N, H), jnp.float32),
    )(gacc, bg)


def _gat_sc_full(x, W, a_src, a_dst, bg, idx_gat, zrows_hw):
    aug, a_d3 = _gat_build_tc(x, W, a_src, a_dst)
    a_dp = jnp.pad(a_d3.reshape(HEADS, N), ((0, 0), (0, NJUNK)))
    ad8 = jnp.pad(a_dp.reshape(HEADS * NACC, 1), ((0, 0), (0, 7)))
    gacc = _gat_sc(aug.reshape(HEADS * N, HW), ad8, idx_gat, zrows_hw)
    return _gat_finish_tc(gacc, bg)


_GBN = 1000


def _gin_block_kernel(h_ref, p_ref, w1_ref, b1_ref, w2_ref, b2_ref,
                      g_ref, bb_ref, out_ref):
    h = h_ref[...]
    agg = p_ref[0] + p_ref[1]
    t = h + agg
    t = jnp.maximum(t @ w1_ref[...] + b1_ref[...], 0.0)
    t = h + (t @ w2_ref[...] + b2_ref[...])
    m = t.mean(-1, keepdims=True)
    v = ((t - m) ** 2).mean(-1, keepdims=True)
    out_ref[...] = (t - m) / jnp.sqrt(v + 1e-5) * g_ref[...] + bb_ref[...]


def _gin_block_tc(h, partials, w1, b1, w2, b2, g, b):
    return pl.pallas_call(
        _gin_block_kernel,
        grid=(N // _GBN,),
        in_specs=[
            pl.BlockSpec((_GBN, H), lambda i: (i, 0)),
            pl.BlockSpec((NCORE, _GBN, H), lambda i: (0, i, 0)),
            pl.BlockSpec((H, H), lambda i: (0, 0)),
            pl.BlockSpec((H,), lambda i: (0,)),
            pl.BlockSpec((H, H), lambda i: (0, 0)),
            pl.BlockSpec((H,), lambda i: (0,)),
            pl.BlockSpec((H,), lambda i: (0,)),
            pl.BlockSpec((H,), lambda i: (0,)),
        ],
        out_specs=pl.BlockSpec((_GBN, H), lambda i: (i, 0)),
        out_shape=jax.ShapeDtypeStruct((N, H), jnp.float32),
    )(h, partials, w1, b1, w2, b2, g, b)


_BN = 1000


def _pool_kernel(h2_ref, bat_ref, gw1_ref, gb1_ref, gw2_ref, gb2_ref,
                 pooled_ref, s_ref):
    i = pl.program_id(0)

    h2 = h2_ref[...]
    g = jnp.maximum(h2 @ gw1_ref[...] + gb1_ref[...], 0.0)
    logit = g @ gw2_ref[...] + gb2_ref[...]
    e = jnp.exp(logit[:, 0])

    bat = bat_ref[...]
    onehot = (bat == lax.broadcasted_iota(jnp.int32, (1, B), 1)).astype(
        jnp.float32)

    pooled_blk = onehot.T @ (e[:, None] * h2)
    s_blk = onehot.T @ e[:, None]

    @pl.when(i == 0)
    def _():
        pooled_ref[...] = jnp.zeros_like(pooled_ref)
        s_ref[...] = jnp.zeros_like(s_ref)

    pooled_ref[...] += pooled_blk
    s_ref[...] += s_blk


def _final_kernel(pooled_ref, s_ref, l1w_ref, l1b_ref, lnfg_ref, lnfb_ref,
                  l2w_ref, l2b_ref, out_ref):
    pooled = pooled_ref[...] / (s_ref[...] + 1e-16)
    z = pooled @ l1w_ref[...] + l1b_ref[...]
    m = z.mean(-1, keepdims=True)
    v = ((z - m) ** 2).mean(-1, keepdims=True)
    z = (z - m) / jnp.sqrt(v + 1e-5) * lnfg_ref[...] + lnfb_ref[...]
    z = jnp.maximum(z, 0.0)
    out_ref[...] = z @ l2w_ref[...] + l2b_ref[...]


def _pool_and_head(h2, batch, gw1, gb1, gw2, gb2, l1w, l1b, lnfg, lnfb,
                   l2w, l2b):
    bat2 = batch[:, None]
    pooled, s = pl.pallas_call(
        _pool_kernel,
        grid=(N // _BN,),
        in_specs=[
            pl.BlockSpec((_BN, H), lambda i: (i, 0)),
            pl.BlockSpec((_BN, 1), lambda i: (i, 0)),
            pl.BlockSpec((H, H), lambda i: (0, 0)),
            pl.BlockSpec((H,), lambda i: (0,)),
            pl.BlockSpec((H, 1), lambda i: (0, 0)),
            pl.BlockSpec((1,), lambda i: (0,)),
        ],
        out_specs=[
            pl.BlockSpec((B, H), lambda i: (0, 0)),
            pl.BlockSpec((B, 1), lambda i: (0, 0)),
        ],
        out_shape=[
            jax.ShapeDtypeStruct((B, H), jnp.float32),
            jax.ShapeDtypeStruct((B, 1), jnp.float32),
        ],
    )(h2, bat2, gw1, gb1, gw2, gb2)

    out = pl.pallas_call(
        _final_kernel,
        out_shape=jax.ShapeDtypeStruct((B, 6), jnp.float32),
    )(pooled, s, l1w, l1b, lnfg, lnfb, l2w, l2b)
    return out


def kernel(x, edge_index, batch, W1, as1, ad1, bg1, g1w1, g1b1, g1w2, g1b2,
           ln1g, ln1b, W2, as2, ad2, bg2, g2w1, g2b1, g2w2, g2b2, ln2g, ln2b,
           gw1, gb1, gw2, gb2, l1w, l1b, lnfg, lnfb, l2w, l2b):
    src0, dst0 = edge_index[0], edge_index[1]
    loop = jnp.arange(N, dtype=edge_index.dtype)
    src = jnp.concatenate([src0, loop])
    dst = jnp.concatenate([dst0, loop])

    e_pad = NT * CH * math.ceil(E / (NT * CH))
    srcp = _pad_edges(src0, e_pad, None)
    dstp = _pad_edges(dst0, e_pad, N)
    idx_gin = jnp.stack([srcp, dstp], axis=2)
    zrows = jnp.zeros((ZROWS, H), jnp.float32)
    zrows_hw = jnp.zeros((ZROWS, HW), jnp.float32)

    eg_pad = NSUB * CH * math.ceil((E + N) / (NSUB * CH))
    npad_g = eg_pad - (E + N)
    src_g = jnp.concatenate(
        [src, (jnp.arange(npad_g, dtype=jnp.int32) * 97) % N])
    dst_g = jnp.concatenate(
        [dst, N + (jnp.arange(npad_g, dtype=jnp.int32) * 97) % NJUNK])
    src_t = src_g.reshape(NSUB, -1, CH)
    dst_t = dst_g.reshape(NSUB, -1, CH)
    idx_gat = jnp.stack(
        [jnp.stack([src_t + hd * N, dst_t + hd * NACC, dst_t], axis=2)
         for hd in range(HEADS)], axis=0)

    h = _gat_sc_full(x, W1, as1, ad1, bg1, idx_gat, zrows_hw)
    h = _gin_block_tc(h, _gin_sc_agg(h, idx_gin, zrows),
                      g1w1, g1b1, g1w2, g1b2, ln1g, ln1b)
    h2 = _gat_sc_full(h, W2, as2, ad2, bg2, idx_gat, zrows_hw)
    h2 = _gin_block_tc(h2, _gin_sc_agg(h2, idx_gin, zrows),
                       g2w1, g2b1, g2w2, g2b2, ln2g, ln2b)

    return _pool_and_head(h2, batch, gw1, gb1, gw2, gb2, l1w, l1b,
                          lnfg, lnfb, l2w, l2b)
